# Initial kernel scaffold; baseline (speedup 1.0000x reference)
import jax
import jax.numpy as jnp
from jax import lax
from jax.experimental import pallas as pl
from jax.experimental.pallas import tpu as pltpu
from jax.experimental.pallas import tpu_sc as plsc

E = 320000
D = 128
L = 16
NC, NS = 2, 16
NW = NC * NS
E_PER_W = E // NW
C = 80
N_CHUNKS = E_PER_W // C
G = C // L


def _sc_body(hd, hs, didx, sidx, out, didx_v, sidx_v, a_v, b_v, tr_v,
             out_v, sem_a, sem_b):
    wid = lax.axis_index("s") * NC + lax.axis_index("c")
    base = wid * E_PER_W

    pltpu.sync_copy(didx.at[pl.ds(base, E_PER_W)], didx_v)
    pltpu.sync_copy(sidx.at[pl.ds(base, E_PER_W)], sidx_v)

    lanes = lax.iota(jnp.int32, L)

    def chunk_body(i, _):
        off = i * C
        cp_a = pltpu.async_copy(hd.at[didx_v.at[pl.ds(off, C)]], a_v, sem_a)
        cp_b = pltpu.async_copy(hs.at[sidx_v.at[pl.ds(off, C)]], b_v, sem_b)
        cp_a.wait()
        cp_b.wait()

        def group_body(g, _):
            e0 = g * L
            for l in range(L):
                e = e0 + l
                acc = a_v[e, pl.ds(0, L)] * b_v[e, pl.ds(0, L)]
                for j in range(1, D // L):
                    acc = acc + a_v[e, pl.ds(j * L, L)] * b_v[e, pl.ds(j * L, L)]
                plsc.store_scatter(tr_v, [lanes, jnp.full((L,), l, jnp.int32)],
                                   acc)
            s = tr_v[0, :]
            for d in range(1, L):
                s = s + tr_v[d, :]
            out_v[pl.ds(e0, L)] = s
            return ()

        lax.fori_loop(0, G, group_body, (), unroll=False)
        pltpu.sync_copy(out_v, out.at[pl.ds(base + off, C)])
        return ()

    lax.fori_loop(0, N_CHUNKS, chunk_body, (), unroll=False)


@jax.jit
def _link_predict(h_drug, h_disease, d_idx, dis_idx):
    mesh = plsc.VectorSubcoreMesh(core_axis_name="c", subcore_axis_name="s",
                                  num_cores=NC, num_subcores=NS)
    return pl.kernel(
        _sc_body,
        out_type=jax.ShapeDtypeStruct((E,), jnp.float32),
        mesh=mesh,
        scratch_types=[
            pltpu.VMEM((E_PER_W,), jnp.int32),
            pltpu.VMEM((E_PER_W,), jnp.int32),
            pltpu.VMEM((C, D), jnp.float32),
            pltpu.VMEM((C, D), jnp.float32),
            pltpu.VMEM((L, L), jnp.float32),
            pltpu.VMEM((C,), jnp.float32),
            pltpu.SemaphoreType.DMA,
            pltpu.SemaphoreType.DMA,
        ],
    )(h_drug, h_disease, d_idx, dis_idx)


def kernel(h_drug, h_disease, edges):
    e32 = edges.astype(jnp.int32)
    return _link_predict(h_drug, h_disease, e32[:, 0], e32[:, 1])


# speedup vs baseline: 3.1713x; 3.1713x over previous
import jax
import jax.numpy as jnp
from jax import lax
from jax.experimental import pallas as pl
from jax.experimental.pallas import tpu as pltpu
from jax.experimental.pallas import tpu_sc as plsc

E = 320000
D = 128
L = 16
NC, NS = 2, 16
NW = NC * NS
E_PER_W = E // NW
C = 80
N_CHUNKS = E_PER_W // C
G = C // L


def _sc_body(hd, hs, didx, sidx, out, didx_v, sidx_v, a_v, b_v,
             out_v, sem_a, sem_b):
    wid = lax.axis_index("s") * NC + lax.axis_index("c")
    base = wid * E_PER_W

    pltpu.sync_copy(didx.at[pl.ds(base, E_PER_W)], didx_v)
    pltpu.sync_copy(sidx.at[pl.ds(base, E_PER_W)], sidx_v)

    lanes = lax.iota(jnp.int32, L)

    def chunk_body(i, _):
        off = i * C
        cp_a = pltpu.async_copy(hd.at[didx_v.at[pl.ds(off, C)]], a_v, sem_a)
        cp_b = pltpu.async_copy(hs.at[sidx_v.at[pl.ds(off, C)]], b_v, sem_b)
        cp_a.wait()
        cp_b.wait()

        def group_body(g, _):
            e0 = g * L
            vec = jnp.zeros((L,), jnp.float32)
            for l in range(L):
                e = e0 + l
                acc = a_v[e, pl.ds(0, L)] * b_v[e, pl.ds(0, L)]
                for j in range(1, D // L):
                    acc = acc + a_v[e, pl.ds(j * L, L)] * b_v[e, pl.ds(j * L, L)]
                s = lax.reduce_sum(acc, axes=(0,))
                vec = jnp.where(lanes == l, s, vec)
            out_v[pl.ds(e0, L)] = vec
            return ()

        lax.fori_loop(0, G, group_body, (), unroll=False)
        pltpu.sync_copy(out_v, out.at[pl.ds(base + off, C)])
        return ()

    lax.fori_loop(0, N_CHUNKS, chunk_body, (), unroll=False)


@jax.jit
def _link_predict(h_drug, h_disease, d_idx, dis_idx):
    mesh = plsc.VectorSubcoreMesh(core_axis_name="c", subcore_axis_name="s",
                                  num_cores=NC, num_subcores=NS)
    return pl.kernel(
        _sc_body,
        out_type=jax.ShapeDtypeStruct((E,), jnp.float32),
        mesh=mesh,
        compiler_params=pltpu.CompilerParams(needs_layout_passes=False),
        scratch_types=[
            pltpu.VMEM((E_PER_W,), jnp.int32),
            pltpu.VMEM((E_PER_W,), jnp.int32),
            pltpu.VMEM((C, D), jnp.float32),
            pltpu.VMEM((C, D), jnp.float32),
            pltpu.VMEM((C,), jnp.float32),
            pltpu.SemaphoreType.DMA,
            pltpu.SemaphoreType.DMA,
        ],
    )(h_drug, h_disease, d_idx, dis_idx)


def kernel(h_drug, h_disease, edges):
    e32 = edges.astype(jnp.int32)
    return _link_predict(h_drug, h_disease, e32[:, 0], e32[:, 1])


# speedup vs baseline: 4.5977x; 1.4498x over previous
import jax
import jax.numpy as jnp
from jax import lax
from jax.experimental import pallas as pl
from jax.experimental.pallas import tpu as pltpu
from jax.experimental.pallas import tpu_sc as plsc

E = 320000
D = 128
L = 16
NC, NS = 2, 16
NW = NC * NS
E_PER_W = E // NW
C = 80
N_CHUNKS = E_PER_W // C
G = C // L


def _sc_body(hd, hs, didx, sidx, out, didx_v, sidx_v, a_v, b_v,
             out_v, sem_a, sem_b, sem_o):
    wid = lax.axis_index("s") * NC + lax.axis_index("c")
    base = wid * E_PER_W

    pltpu.sync_copy(didx.at[pl.ds(base, E_PER_W)], didx_v)
    pltpu.sync_copy(sidx.at[pl.ds(base, E_PER_W)], sidx_v)

    lanes = lax.iota(jnp.int32, L)

    def issue_gathers(i, slot):
        off = i * C
        pltpu.async_copy(hd.at[didx_v.at[pl.ds(off, C)]], a_v.at[slot],
                         sem_a.at[slot])
        pltpu.async_copy(hs.at[sidx_v.at[pl.ds(off, C)]], b_v.at[slot],
                         sem_b.at[slot])

    issue_gathers(0, 0)

    def chunk_body(i, _):
        slot = lax.rem(i, 2)
        off = i * C

        @pl.when(i + 1 < N_CHUNKS)
        def _():
            issue_gathers(i + 1, lax.rem(i + 1, 2))

        pltpu.make_async_copy(hd.at[didx_v.at[pl.ds(off, C)]], a_v.at[slot],
                              sem_a.at[slot]).wait()
        pltpu.make_async_copy(hs.at[sidx_v.at[pl.ds(off, C)]], b_v.at[slot],
                              sem_b.at[slot]).wait()

        @pl.when(i >= 2)
        def _():
            pltpu.make_async_copy(
                out_v.at[slot], out.at[pl.ds(base + off - 2 * C, C)],
                sem_o.at[slot]).wait()

        def group_body(g, _):
            e0 = g * L
            vec = jnp.zeros((L,), jnp.float32)
            for l in range(L):
                e = e0 + l
                acc = a_v[slot, e, pl.ds(0, L)] * b_v[slot, e, pl.ds(0, L)]
                for j in range(1, D // L):
                    acc = acc + (a_v[slot, e, pl.ds(j * L, L)]
                                 * b_v[slot, e, pl.ds(j * L, L)])
                s = lax.reduce_sum(acc, axes=(0,))
                vec = jnp.where(lanes == l, s, vec)
            out_v[slot, pl.ds(e0, L)] = vec
            return ()

        lax.fori_loop(0, G, group_body, (), unroll=False)
        pltpu.async_copy(out_v.at[slot], out.at[pl.ds(base + off, C)],
                         sem_o.at[slot])
        return ()

    lax.fori_loop(0, N_CHUNKS, chunk_body, (), unroll=False)

    for k in (N_CHUNKS - 2, N_CHUNKS - 1):
        pltpu.make_async_copy(out_v.at[k % 2],
                              out.at[pl.ds(base + k * C, C)],
                              sem_o.at[k % 2]).wait()


@jax.jit
def _link_predict(h_drug, h_disease, d_idx, dis_idx):
    mesh = plsc.VectorSubcoreMesh(core_axis_name="c", subcore_axis_name="s",
                                  num_cores=NC, num_subcores=NS)
    return pl.kernel(
        _sc_body,
        out_type=jax.ShapeDtypeStruct((E,), jnp.float32),
        mesh=mesh,
        compiler_params=pltpu.CompilerParams(needs_layout_passes=False),
        scratch_types=[
            pltpu.VMEM((E_PER_W,), jnp.int32),
            pltpu.VMEM((E_PER_W,), jnp.int32),
            pltpu.VMEM((2, C, D), jnp.float32),
            pltpu.VMEM((2, C, D), jnp.float32),
            pltpu.VMEM((2, C), jnp.float32),
            pltpu.SemaphoreType.DMA((2,)),
            pltpu.SemaphoreType.DMA((2,)),
            pltpu.SemaphoreType.DMA((2,)),
        ],
    )(h_drug, h_disease, d_idx, dis_idx)


def kernel(h_drug, h_disease, edges):
    e32 = edges.astype(jnp.int32)
    return _link_predict(h_drug, h_disease, e32[:, 0], e32[:, 1])


# speedup vs baseline: 8.7133x; 1.8951x over previous
import jax
import jax.numpy as jnp
from jax import lax
from jax.experimental import pallas as pl
from jax.experimental.pallas import tpu as pltpu
from jax.experimental.pallas import tpu_sc as plsc

E = 320000
D = 128
L = 16
NC, NS = 2, 16
NW = NC * NS
E_PER_W = E // NW
C = 80
N_CHUNKS = E_PER_W // C
G = C // L


def _sc_body(hd, hs, didx, sidx, out, didx_v, sidx_v, a_v, b_v,
             out_v, sem_a, sem_b, sem_o):
    wid = lax.axis_index("s") * NC + lax.axis_index("c")
    base = wid * E_PER_W

    pltpu.sync_copy(didx.at[pl.ds(base, E_PER_W)], didx_v)
    pltpu.sync_copy(sidx.at[pl.ds(base, E_PER_W)], sidx_v)

    lanes = lax.iota(jnp.int32, L)

    def issue_gathers(i, slot):
        off = i * C
        pltpu.async_copy(hd.at[didx_v.at[pl.ds(off, C)]], a_v.at[slot],
                         sem_a.at[slot])
        pltpu.async_copy(hs.at[sidx_v.at[pl.ds(off, C)]], b_v.at[slot],
                         sem_b.at[slot])

    issue_gathers(0, 0)

    def chunk_body(i, _):
        slot = lax.rem(i, 2)
        off = i * C

        @pl.when(i + 1 < N_CHUNKS)
        def _():
            issue_gathers(i + 1, lax.rem(i + 1, 2))

        pltpu.make_async_copy(hd.at[didx_v.at[pl.ds(off, C)]], a_v.at[slot],
                              sem_a.at[slot]).wait()
        pltpu.make_async_copy(hs.at[sidx_v.at[pl.ds(off, C)]], b_v.at[slot],
                              sem_b.at[slot]).wait()

        @pl.when(i >= 2)
        def _():
            pltpu.make_async_copy(
                out_v.at[slot], out.at[pl.ds(base + off - 2 * C, C)],
                sem_o.at[slot]).wait()

        def group_body(g, _):
            e0 = g * L
            vec = jnp.zeros((L,), jnp.float32)
            for l in range(L):
                e = e0 + l
                acc = jnp.zeros((L,), jnp.float32)
                for j in range(D // (2 * L)):
                    a2 = plsc.bitcast(a_v[slot, e, pl.ds(j * L, L)],
                                      jnp.bfloat16)
                    b2 = plsc.bitcast(b_v[slot, e, pl.ds(j * L, L)],
                                      jnp.bfloat16)
                    p0, p1 = plsc.unpack(
                        a2 * b2, format=plsc.PackFormat.INTERLEAVED,
                        preferred_element_type=jnp.float32)
                    acc = acc + p0 + p1
                s = lax.reduce_sum(acc, axes=(0,))
                vec = jnp.where(lanes == l, s, vec)
            out_v[slot, pl.ds(e0, L)] = vec
            return ()

        lax.fori_loop(0, G, group_body, (), unroll=False)
        pltpu.async_copy(out_v.at[slot], out.at[pl.ds(base + off, C)],
                         sem_o.at[slot])
        return ()

    lax.fori_loop(0, N_CHUNKS, chunk_body, (), unroll=False)

    for k in (N_CHUNKS - 2, N_CHUNKS - 1):
        pltpu.make_async_copy(out_v.at[k % 2],
                              out.at[pl.ds(base + k * C, C)],
                              sem_o.at[k % 2]).wait()


@jax.jit
def _link_predict(h_drug, h_disease, d_idx, dis_idx):
    mesh = plsc.VectorSubcoreMesh(core_axis_name="c", subcore_axis_name="s",
                                  num_cores=NC, num_subcores=NS)
    return pl.kernel(
        _sc_body,
        out_type=jax.ShapeDtypeStruct((E,), jnp.float32),
        mesh=mesh,
        compiler_params=pltpu.CompilerParams(needs_layout_passes=False,
                                             use_tc_tiling_on_sc=False),
        scratch_types=[
            pltpu.VMEM((E_PER_W,), jnp.int32),
            pltpu.VMEM((E_PER_W,), jnp.int32),
            pltpu.VMEM((2, C, D // 2), jnp.int32),
            pltpu.VMEM((2, C, D // 2), jnp.int32),
            pltpu.VMEM((2, C), jnp.float32),
            pltpu.SemaphoreType.DMA((2,)),
            pltpu.SemaphoreType.DMA((2,)),
            pltpu.SemaphoreType.DMA((2,)),
        ],
    )(h_drug, h_disease, d_idx, dis_idx)


def _pack_table(h):
    hb = h.astype(jnp.bfloat16).reshape(h.shape[0], h.shape[1] // 2, 2)
    return lax.bitcast_convert_type(hb, jnp.int32)


def kernel(h_drug, h_disease, edges):
    e32 = edges.astype(jnp.int32)
    return _link_predict(_pack_table(h_drug), _pack_table(h_disease),
                         e32[:, 0], e32[:, 1])


# speedup vs baseline: 11.8529x; 1.3603x over previous
import jax
import jax.numpy as jnp
from jax import lax
from jax.experimental import pallas as pl
from jax.experimental.pallas import tpu as pltpu
from jax.experimental.pallas import tpu_sc as plsc

E = 320000
D = 128
L = 16
NC, NS = 2, 16
NW = NC * NS
E_PER_W = E // NW
C = 80
N_CHUNKS = E_PER_W // C
G = C // L


def _sc_body(hd, hs, didx, sidx, out, didx_v, sidx_v, a_v, b_v,
             out_v, sem_a, sem_b, sem_o):
    wid = lax.axis_index("s") * NC + lax.axis_index("c")
    base = wid * E_PER_W

    pltpu.sync_copy(didx.at[pl.ds(base, E_PER_W)], didx_v)
    pltpu.sync_copy(sidx.at[pl.ds(base, E_PER_W)], sidx_v)

    lanes = lax.iota(jnp.int32, L)

    def issue_gathers(i, slot):
        off = i * C
        pltpu.async_copy(hd.at[didx_v.at[pl.ds(off, C)]], a_v.at[slot],
                         sem_a.at[slot])
        pltpu.async_copy(hs.at[sidx_v.at[pl.ds(off, C)]], b_v.at[slot],
                         sem_b.at[slot])

    issue_gathers(0, 0)

    def chunk_body(i, _):
        slot = lax.rem(i, 2)
        off = i * C

        @pl.when(i + 1 < N_CHUNKS)
        def _():
            issue_gathers(i + 1, lax.rem(i + 1, 2))

        pltpu.make_async_copy(hd.at[didx_v.at[pl.ds(off, C)]], a_v.at[slot],
                              sem_a.at[slot]).wait()
        pltpu.make_async_copy(hs.at[sidx_v.at[pl.ds(off, C)]], b_v.at[slot],
                              sem_b.at[slot]).wait()

        @pl.when(i >= 2)
        def _():
            pltpu.make_async_copy(
                out_v.at[slot], out.at[pl.ds(base + off - 2 * C, C)],
                sem_o.at[slot]).wait()

        def group_body(g, _):
            e0 = g * L
            vec = jnp.zeros((L,), jnp.float32)
            for l in range(L):
                e = e0 + l
                acc = jnp.zeros((L,), jnp.float32)
                for j in range(D // (2 * L)):
                    a2 = plsc.bitcast(a_v[slot, e, pl.ds(j * L, L)],
                                      jnp.bfloat16)
                    b2 = plsc.bitcast(b_v[slot, e, pl.ds(j * L, L)],
                                      jnp.bfloat16)
                    p0, p1 = plsc.unpack(
                        a2 * b2, format=plsc.PackFormat.INTERLEAVED,
                        preferred_element_type=jnp.float32)
                    acc = acc + p0 + p1
                s = lax.reduce_sum(acc, axes=(0,))
                vec = jnp.where(lanes == l, s, vec)
            out_v[slot, pl.ds(e0, L)] = vec
            return ()

        lax.fori_loop(0, G, group_body, (), unroll=False)
        pltpu.async_copy(out_v.at[slot], out.at[pl.ds(base + off, C)],
                         sem_o.at[slot])
        return ()

    lax.fori_loop(0, N_CHUNKS, chunk_body, (), unroll=False)

    for k in (N_CHUNKS - 2, N_CHUNKS - 1):
        pltpu.make_async_copy(out_v.at[k % 2],
                              out.at[pl.ds(base + k * C, C)],
                              sem_o.at[k % 2]).wait()


@jax.jit
def _link_predict(h_drug, h_disease, d_idx, dis_idx):
    mesh = plsc.VectorSubcoreMesh(core_axis_name="c", subcore_axis_name="s",
                                  num_cores=NC, num_subcores=NS)
    return pl.kernel(
        _sc_body,
        out_type=jax.ShapeDtypeStruct((E,), jnp.float32),
        mesh=mesh,
        compiler_params=pltpu.CompilerParams(needs_layout_passes=False,
                                             use_tc_tiling_on_sc=False),
        scratch_types=[
            pltpu.VMEM((E_PER_W,), jnp.int32),
            pltpu.VMEM((E_PER_W,), jnp.int32),
            pltpu.VMEM((2, C, D // 2), jnp.int32),
            pltpu.VMEM((2, C, D // 2), jnp.int32),
            pltpu.VMEM((2, C), jnp.float32),
            pltpu.SemaphoreType.DMA((2,)),
            pltpu.SemaphoreType.DMA((2,)),
            pltpu.SemaphoreType.DMA((2,)),
        ],
    )(h_drug, h_disease, d_idx, dis_idx)


def _pack_table(h):
    u = lax.bitcast_convert_type(h, jnp.uint32)
    lo, hi = u[:, : D // 2], u[:, D // 2 :]
    half = jnp.uint32(0x7FFF)
    rlo = lo + half + ((lo >> 16) & jnp.uint32(1))
    rhi = hi + half + ((hi >> 16) & jnp.uint32(1))
    packed = (rhi & jnp.uint32(0xFFFF0000)) | (rlo >> 16)
    return lax.bitcast_convert_type(packed, jnp.int32)


def kernel(h_drug, h_disease, edges):
    e32 = edges.astype(jnp.int32)
    return _link_predict(_pack_table(h_drug), _pack_table(h_disease),
                         e32[:, 0], e32[:, 1])
